# Initial kernel scaffold; baseline (speedup 1.0000x reference)
import jax
import jax.numpy as jnp
from jax.experimental import pallas as pl


def kernel(x, edge_index, Wl0, b0, Wr0, Wl1, b1, Wr1, Wlin, blin):
    raise NotImplementedError("write your pallas kernel here")


# speedup vs baseline: 5.7472x; 5.7472x over previous
import functools

import jax
import jax.numpy as jnp
from jax import lax
from jax.experimental import pallas as pl
from jax.experimental.pallas import tpu as pltpu
from jax.experimental.pallas import tpu_sc as plsc

N = 10000
E = 320000
D = 128
NC = 2
NS = 16

EPC = E // NC
EPT = EPC // NS
G = 128
GROUPS = 2
CHUNK = G * GROUPS
NCHUNK = EPT // CHUNK
REM = EPT - NCHUNK * CHUNK

EPW = E // (NC * NS)
DCHUNK = 400
NDCHUNK = EPW // DCHUNK

ROWS_PT = 624
TAIL = N - ROWS_PT * NS


def _sc_agg_body(with_hist, *refs):
    if with_hist:
        (tbl_h, src_h, dst_h, agg_h, hists_h,
         agg_sp, ibuf, dst2, srcr, dstr, rows, hist,
         isem, gsem, ssem) = refs
    else:
        (tbl_h, src_h, dst_h, agg_h,
         agg_sp, ibuf, dst2, srcr, dstr, rows,
         isem, gsem, ssem) = refs
        hist = None

    cid = lax.axis_index("c")
    sid = lax.axis_index("s")

    zeros16 = jnp.zeros((16,), jnp.float32)

    def zrow(i, c):
        for j in range(D // 16):
            rows[i, pl.ds(j * 16, 16)] = zeros16
        return c

    lax.fori_loop(0, 128, zrow, 0)
    zb = sid * ROWS_PT
    for k in range(4):
        pltpu.sync_copy(rows.at[pl.ds(0, 128)],
                        agg_sp.at[pl.ds(zb + k * 128, 128)])
    pltpu.sync_copy(rows.at[pl.ds(0, ROWS_PT - 512)],
                    agg_sp.at[pl.ds(zb + 512, ROWS_PT - 512)])

    @pl.when(sid == 0)
    def _():
        pltpu.sync_copy(rows.at[pl.ds(0, TAIL)],
                        agg_sp.at[pl.ds(ROWS_PT * NS, TAIL)])

    if hist is not None:
        def zh(i, c):
            hist[pl.ds(i * 16, 16)] = zeros16
            return c

        lax.fori_loop(0, N // 16, zh, 0)
        wid = cid * NS + sid
        dbase = wid * EPW
        ones16 = jnp.full((16,), 1.0, jnp.float32)

        def dchunk(q, c):
            pltpu.sync_copy(dst_h.at[pl.ds(dbase + q * DCHUNK, DCHUNK)],
                            ibuf.at[pl.ds(0, DCHUNK)])
            for j in range(DCHUNK // 16):
                v = ibuf[pl.ds(j * 16, 16)]
                plsc.addupdate_scatter(hist, [v], ones16)
            return c

        lax.fori_loop(0, NDCHUNK, dchunk, 0)
        pltpu.sync_copy(hist, hists_h.at[wid, 0])

    plsc.subcore_barrier()

    tile_base = cid * EPC + sid * EPT

    def chunk(g, c):
        base = tile_base + g * CHUNK
        pltpu.sync_copy(src_h.at[pl.ds(base, CHUNK)],
                        ibuf.at[pl.ds(0, CHUNK)])
        dws = [pltpu.async_copy(dst_h.at[pl.ds(base + k * G, G)],
                                dst2.at[k], isem) for k in range(GROUPS)]
        for d in dws:
            d.wait()
        gws = [pltpu.async_copy(tbl_h.at[ibuf.at[pl.ds(k * G, G)]],
                                rows.at[pl.ds(k * G, G)], gsem)
               for k in range(GROUPS)]
        for d in gws:
            d.wait()
        sws = [pltpu.async_copy(rows.at[pl.ds(k * G, G)],
                                agg_sp.at[dst2.at[k]], ssem, add=True)
               for k in range(GROUPS)]
        for d in sws:
            d.wait()
        return c

    lax.fori_loop(0, NCHUNK, chunk, 0)

    rbase = tile_base + NCHUNK * CHUNK
    pltpu.sync_copy(src_h.at[pl.ds(rbase, REM)], srcr)
    pltpu.sync_copy(dst_h.at[pl.ds(rbase, REM)], dstr)
    pltpu.async_copy(tbl_h.at[srcr], rows.at[pl.ds(0, REM)], gsem).wait()
    pltpu.async_copy(rows.at[pl.ds(0, REM)], agg_sp.at[dstr], ssem,
                     add=True).wait()

    plsc.subcore_barrier()

    ob = sid * ROWS_PT
    pltpu.sync_copy(agg_sp.at[pl.ds(ob, ROWS_PT)],
                    agg_h.at[cid, pl.ds(ob, ROWS_PT)])

    @pl.when(sid == 0)
    def _():
        pltpu.sync_copy(agg_sp.at[pl.ds(ROWS_PT * NS, TAIL)],
                        agg_h.at[cid, pl.ds(ROWS_PT * NS, TAIL)])


def _make_sc_agg(with_hist):
    out_type = [jax.ShapeDtypeStruct((NC, N, D), jnp.float32)]
    if with_hist:
        out_type.append(jax.ShapeDtypeStruct((NC * NS, 1, N), jnp.float32))
    scratch = [
        pltpu.VMEM_SHARED((N, D), jnp.float32),
        pltpu.VMEM((DCHUNK,), jnp.int32),
        pltpu.VMEM((GROUPS, G), jnp.int32),
        pltpu.VMEM((REM,), jnp.int32),
        pltpu.VMEM((REM,), jnp.int32),
        pltpu.VMEM((CHUNK, D), jnp.float32),
    ]
    if with_hist:
        scratch.append(pltpu.VMEM((N,), jnp.float32))
    scratch += [pltpu.SemaphoreType.DMA] * 3
    return pl.kernel(
        functools.partial(_sc_agg_body, with_hist),
        out_type=tuple(out_type),
        mesh=plsc.VectorSubcoreMesh(core_axis_name="c", subcore_axis_name="s"),
        scratch_types=scratch,
        compiler_params=pltpu.CompilerParams(needs_layout_passes=False),
    )


BLK = 1000
NBLK = N // BLK


def _tc_project_body(x_ref, w_ref, out_ref):
    out_ref[...] = jnp.dot(x_ref[...], w_ref[...],
                           preferred_element_type=jnp.float32)


def _tc_project(x, w):
    return pl.pallas_call(
        _tc_project_body,
        grid=(NBLK,),
        in_specs=[pl.BlockSpec((BLK, D), lambda i: (i, 0)),
                  pl.BlockSpec((D, D), lambda i: (0, 0))],
        out_specs=pl.BlockSpec((BLK, D), lambda i: (i, 0)),
        out_shape=jax.ShapeDtypeStruct((N, D), jnp.float32),
    )(x, w)


def _inv_deg(hists):
    deg = jnp.sum(hists, axis=0)
    return 1.0 / jnp.maximum(deg, 1.0)


def _tc_mid_body(agg_ref, h_ref, x_ref, wr0_ref, b0_ref, wl1_ref, wr1_ref,
                 p1_ref, r1_ref):
    inv = _inv_deg(h_ref[...])
    agg = agg_ref[0] + agg_ref[1]
    mean = agg * inv
    h = jnp.maximum(
        mean + b0_ref[...]
        + jnp.dot(x_ref[...], wr0_ref[...], preferred_element_type=jnp.float32),
        0.0)
    p1_ref[...] = jnp.dot(h, wl1_ref[...], preferred_element_type=jnp.float32)
    r1_ref[...] = jnp.dot(h, wr1_ref[...], preferred_element_type=jnp.float32)


def _tc_mid(agg0, hists, x, wr0, b0, wl1, wr1):
    return pl.pallas_call(
        _tc_mid_body,
        grid=(NBLK,),
        in_specs=[pl.BlockSpec((NC, BLK, D), lambda i: (0, i, 0)),
                  pl.BlockSpec((NC * NS, BLK, 1), lambda i: (0, i, 0)),
                  pl.BlockSpec((BLK, D), lambda i: (i, 0)),
                  pl.BlockSpec((D, D), lambda i: (0, 0)),
                  pl.BlockSpec((1, D), lambda i: (0, 0)),
                  pl.BlockSpec((D, D), lambda i: (0, 0)),
                  pl.BlockSpec((D, D), lambda i: (0, 0))],
        out_specs=[pl.BlockSpec((BLK, D), lambda i: (i, 0)),
                   pl.BlockSpec((BLK, D), lambda i: (i, 0))],
        out_shape=[jax.ShapeDtypeStruct((N, D), jnp.float32),
                   jax.ShapeDtypeStruct((N, D), jnp.float32)],
    )(agg0, hists, x, wr0, b0, wl1, wr1)


DO = 64


def _tc_final_body(agg_ref, h_ref, r1_ref, b1_ref, wlin_ref, blin_ref,
                   out_ref):
    inv = _inv_deg(h_ref[...])
    agg = agg_ref[0] + agg_ref[1]
    h2 = jnp.maximum(agg * inv + b1_ref[...] + r1_ref[...], 0.0)
    o = jnp.dot(h2, wlin_ref[...], preferred_element_type=jnp.float32)
    o = o + blin_ref[...]
    m = jnp.max(o, axis=1, keepdims=True)
    e = jnp.exp(o - m)
    out_ref[...] = e / jnp.sum(e, axis=1, keepdims=True)


def _tc_final(agg1, hists, r1, b1, wlin, blin):
    return pl.pallas_call(
        _tc_final_body,
        grid=(NBLK,),
        in_specs=[pl.BlockSpec((NC, BLK, D), lambda i: (0, i, 0)),
                  pl.BlockSpec((NC * NS, BLK, 1), lambda i: (0, i, 0)),
                  pl.BlockSpec((BLK, D), lambda i: (i, 0)),
                  pl.BlockSpec((1, D), lambda i: (0, 0)),
                  pl.BlockSpec((D, DO), lambda i: (0, 0)),
                  pl.BlockSpec((1, DO), lambda i: (0, 0))],
        out_specs=pl.BlockSpec((BLK, DO), lambda i: (i, 0)),
        out_shape=jax.ShapeDtypeStruct((N, DO), jnp.float32),
    )(agg1, hists, r1, b1, wlin, blin)


_sc_agg_hist = _make_sc_agg(True)
_sc_agg = _make_sc_agg(False)


def kernel(x, edge_index, Wl0, b0, Wr0, Wl1, b1, Wr1, Wlin, blin):
    src = edge_index[0]
    dst = edge_index[1]

    p0 = _tc_project(x, Wl0)
    agg0, hists = _sc_agg_hist(p0, src, dst)
    hists = hists.reshape(NC * NS, N, 1)
    p1, r1 = _tc_mid(agg0, hists, x, Wr0, b0.reshape(1, D), Wl1, Wr1)
    agg1, = _sc_agg(p1, src, dst)
    out = _tc_final(agg1, hists, r1, b1.reshape(1, D), Wlin,
                    blin.reshape(1, DO))
    return out
